# baseline (speedup 1.0000x reference)
import functools

import jax
import jax.numpy as jnp
from jax import lax
from jax.experimental import pallas as pl
from jax.experimental.pallas import tpu as pltpu
from jax.experimental.pallas import tpu_sc as plsc

N = 10000
F = 128
E = 320000

NC = 2
NS = 16
NW = NC * NS

N_PAD = 10240
ROWS_PER_TILE = N_PAD // NS

CH = 128
CH4 = CH // 4
NCHUNK = 80
EW = NCHUNK * CH
E_PAD = EW * NW
SEG = 8
NSEG = NCHUNK // SEG

_mesh = plsc.VectorSubcoreMesh(
    core_axis_name="c", subcore_axis_name="s", num_cores=NC, num_subcores=NS
)

ZROWS = 8


def _zero_vmem_2d(buf, rows):
    def body(i, _):
        for k in range(F // 16):
            buf[i, pl.ds(k * 16, 16)] = jnp.zeros((16,), jnp.float32)
        return 0

    lax.fori_loop(0, rows, body, 0)


@functools.partial(
    pl.kernel,
    out_type=jax.ShapeDtypeStruct((NC, N_PAD), jnp.float32),
    mesh=_mesh,
    scratch_types=[
        pltpu.VMEM((NCHUNK, CH), jnp.int32),
        pltpu.VMEM((CH,), jnp.float32),
        pltpu.VMEM((ROWS_PER_TILE,), jnp.float32),
        pltpu.VMEM_SHARED((N_PAD,), jnp.float32),
        pltpu.SemaphoreType.DMA,
    ],
)
def _deg_kernel(dst_hbm, degp_hbm, idx_v, ones_v, zbuf_v, acc_sh, sem):
    core = lax.axis_index("c")
    sid = lax.axis_index("s")
    wid = core * NS + sid

    for k in range(CH // 16):
        ones_v[pl.ds(k * 16, 16)] = jnp.ones((16,), jnp.float32)

    def zb(i, _):
        zbuf_v[pl.ds(i * 16, 16)] = jnp.zeros((16,), jnp.float32)
        return 0

    lax.fori_loop(0, ROWS_PER_TILE // 16, zb, 0)
    pltpu.sync_copy(zbuf_v, acc_sh.at[pl.ds(sid * ROWS_PER_TILE, ROWS_PER_TILE)])
    pltpu.sync_copy(dst_hbm.at[wid], idx_v)
    plsc.subcore_barrier()

    def fire(c, _):
        pltpu.async_copy(ones_v, acc_sh.at[idx_v.at[c]], sem, add=True)
        return 0

    lax.fori_loop(0, NCHUNK, fire, 0)

    def drain(c, _):
        pltpu.make_async_copy(ones_v, acc_sh.at[pl.ds(0, CH)], sem).wait()
        return 0

    lax.fori_loop(0, NCHUNK, drain, 0)
    plsc.subcore_barrier()

    sl = pl.ds(sid * ROWS_PER_TILE, ROWS_PER_TILE)
    pltpu.sync_copy(acc_sh.at[sl], degp_hbm.at[core, sl])


@functools.partial(
    pl.kernel,
    out_type=jax.ShapeDtypeStruct((NC, N_PAD, F), jnp.float32),
    mesh=_mesh,
    scratch_types=[
        pltpu.VMEM((2, SEG, CH), jnp.int32),
        pltpu.VMEM((NCHUNK, CH), jnp.int32),
        pltpu.VMEM((2, CH, F), jnp.float32),
        pltpu.VMEM((ZROWS, F), jnp.float32),
        pltpu.VMEM_SHARED((N_PAD, F), jnp.float32),
        pltpu.SemaphoreType.DMA,
        pltpu.SemaphoreType.DMA,
        pltpu.SemaphoreType.DMA,
        pltpu.SemaphoreType.DMA,
        pltpu.SemaphoreType.DMA,
        pltpu.SemaphoreType.DMA,
        pltpu.SemaphoreType.DMA,
        pltpu.SemaphoreType.DMA,
        pltpu.SemaphoreType.DMA,
        pltpu.SemaphoreType.DMA,
        pltpu.SemaphoreType.DMA,
        pltpu.SemaphoreType.DMA,
    ],
)
def _agg_kernel(hs_hbm, src_hbm, dst_hbm, accp_hbm,
                sbuf_v, idxd_v, rows_v, zbuf_v, acc_sh,
                g00, g01, g02, g03, g10, g11, g12, g13, s0, s1, i0, i1):
    core = lax.axis_index("c")
    sid = lax.axis_index("s")
    wid = core * NS + sid
    gsem = ((g00, g01, g02, g03), (g10, g11, g12, g13))
    ssem = (s0, s1)
    isem = (i0, i1)

    def issue_i(seg):
        p = seg % 2
        pltpu.async_copy(src_hbm.at[wid, seg], sbuf_v.at[p], isem[p])

    def wait_i(seg):
        p = seg % 2
        pltpu.make_async_copy(src_hbm.at[0, 0], sbuf_v.at[p], isem[p]).wait()

    issue_i(0)
    pltpu.sync_copy(dst_hbm.at[wid], idxd_v)
    _zero_vmem_2d(zbuf_v, ZROWS)

    def zcopy(j, _):
        pltpu.sync_copy(
            zbuf_v, acc_sh.at[pl.ds(sid * ROWS_PER_TILE + j * ZROWS, ZROWS)]
        )
        return 0

    lax.fori_loop(0, ROWS_PER_TILE // ZROWS, zcopy, 0)
    plsc.subcore_barrier()

    def issue_g(c, b):
        seg, l = divmod(c, SEG)
        for q in range(4):
            pltpu.async_copy(
                hs_hbm.at[sbuf_v.at[seg % 2, l, pl.ds(q * CH4, CH4)]],
                rows_v.at[b, pl.ds(q * CH4, CH4)],
                gsem[b][q],
            )

    def wait_g(b):
        for q in range(4):
            pltpu.make_async_copy(
                hs_hbm.at[pl.ds(0, CH4)],
                rows_v.at[b, pl.ds(q * CH4, CH4)],
                gsem[b][q],
            ).wait()

    def issue_s(c, b):
        pltpu.async_copy(rows_v.at[b], acc_sh.at[idxd_v.at[c]], ssem[b], add=True)

    def wait_s(b):
        pltpu.make_async_copy(rows_v.at[b], acc_sh.at[pl.ds(0, CH)], ssem[b]).wait()

    wait_i(0)
    issue_g(0, 0)
    for c in range(NCHUNK):
        b = c % 2
        seg, l = divmod(c, SEG)
        if l == 0 and seg + 1 < NSEG:
            issue_i(seg + 1)
        wait_g(b)
        issue_s(c, b)
        if c > 0:
            wait_s(1 - b)
        if c + 1 < NCHUNK:
            if l == SEG - 1:
                wait_i(seg + 1)
            issue_g(c + 1, 1 - b)
    wait_s(1)

    plsc.subcore_barrier()
    sl = pl.ds(sid * ROWS_PER_TILE, ROWS_PER_TILE)
    pltpu.sync_copy(acc_sh.at[sl], accp_hbm.at[core, sl])


RB = 1280
GRID = N_PAD // RB


def _dinv(degT_ref):
    d = degT_ref[:, 0:1] + degT_ref[:, 1:2] + 1.0
    return lax.rsqrt(d)


def _tc1_body(x_ref, w_ref, degT_ref, o_ref):
    h = jnp.dot(x_ref[...], w_ref[...], preferred_element_type=jnp.float32)
    o_ref[...] = _dinv(degT_ref) * h


def _tc2_body(acc_ref, hs1_ref, degT_ref, b1_ref, w2_ref, o_ref):
    dinv = _dinv(degT_ref)
    t = dinv * (acc_ref[0] + acc_ref[1] + hs1_ref[...]) + b1_ref[...]
    t = jnp.where(t >= 0, t, 0.01 * t)
    o_ref[...] = dinv * jnp.dot(t, w2_ref[...], preferred_element_type=jnp.float32)


def _tc3_body(acc_ref, hs2_ref, degT_ref, b2_ref, o_ref):
    dinv = _dinv(degT_ref)
    o_ref[...] = dinv * (acc_ref[0] + acc_ref[1] + hs2_ref[...]) + b2_ref[...]


_row_spec = pl.BlockSpec((RB, F), lambda i: (i, 0))
_degT_spec = pl.BlockSpec((RB, 2), lambda i: (i, 0))
_w_spec = pl.BlockSpec((F, F), lambda i: (0, 0))
_b_spec = pl.BlockSpec((1, F), lambda i: (0, 0))
_acc_spec = pl.BlockSpec((NC, RB, F), lambda i: (0, i, 0))

_tc1 = pl.pallas_call(
    _tc1_body,
    grid=(GRID,),
    in_specs=[_row_spec, _w_spec, _degT_spec],
    out_specs=_row_spec,
    out_shape=jax.ShapeDtypeStruct((N_PAD, F), jnp.float32),
)

_tc2 = pl.pallas_call(
    _tc2_body,
    grid=(GRID,),
    in_specs=[_acc_spec, _row_spec, _degT_spec, _b_spec, _w_spec],
    out_specs=_row_spec,
    out_shape=jax.ShapeDtypeStruct((N_PAD, F), jnp.float32),
)

_tc3 = pl.pallas_call(
    _tc3_body,
    grid=(GRID,),
    in_specs=[_acc_spec, _row_spec, _degT_spec, _b_spec],
    out_specs=_row_spec,
    out_shape=jax.ShapeDtypeStruct((N_PAD, F), jnp.float32),
)


def kernel(x, edge_index, W1, b1, W2, b2):
    src = edge_index[0].astype(jnp.int32)
    dst = edge_index[1].astype(jnp.int32)

    npad_e = E_PAD - E
    pad_iota = jnp.arange(npad_e, dtype=jnp.int32)
    src_p = jnp.concatenate([src, pad_iota % N])
    dst_p = jnp.concatenate([dst, N + pad_iota % (N_PAD - N)])
    src_p = src_p.reshape(NW, NSEG, SEG, CH)
    dst_p = dst_p.reshape(NW, NCHUNK, CH)
    x_p = jnp.pad(x, ((0, N_PAD - N), (0, 0)))

    degp = _deg_kernel(dst_p)
    degT = degp.T

    b1r = b1.reshape(1, F)
    b2r = b2.reshape(1, F)

    hs1 = _tc1(x_p, W1, degT)
    acc1 = _agg_kernel(hs1, src_p, dst_p)
    hs2 = _tc2(acc1, hs1, degT, b1r, W2)
    acc2 = _agg_kernel(hs2, src_p, dst_p)
    out = _tc3(acc2, hs2, degT, b2r)
    return out[:N]


# speedup vs baseline: 1.0708x; 1.0708x over previous
import functools

import jax
import jax.numpy as jnp
from jax import lax
from jax.experimental import pallas as pl
from jax.experimental.pallas import tpu as pltpu
from jax.experimental.pallas import tpu_sc as plsc

N = 10000
F = 128
E = 320000

NC = 2
NS = 16
NW = NC * NS

N_PAD = 10240
ROWS_PER_TILE = N_PAD // NS

CH = 128
CH2 = CH // 2
NCHUNK = 80
EW = NCHUNK * CH
E_PAD = EW * NW
SEG = 8
NSEG = NCHUNK // SEG

_mesh = plsc.VectorSubcoreMesh(
    core_axis_name="c", subcore_axis_name="s", num_cores=NC, num_subcores=NS
)

ZROWS = 8


def _zero_vmem_2d(buf, rows):
    def body(i, _):
        for k in range(F // 16):
            buf[i, pl.ds(k * 16, 16)] = jnp.zeros((16,), jnp.float32)
        return 0

    lax.fori_loop(0, rows, body, 0)


@functools.partial(
    pl.kernel,
    out_type=jax.ShapeDtypeStruct((NC, N_PAD), jnp.float32),
    mesh=_mesh,
    scratch_types=[
        pltpu.VMEM((NCHUNK, CH), jnp.int32),
        pltpu.VMEM((CH,), jnp.float32),
        pltpu.VMEM((ROWS_PER_TILE,), jnp.float32),
        pltpu.VMEM_SHARED((N_PAD,), jnp.float32),
        pltpu.SemaphoreType.DMA,
    ],
)
def _deg_kernel(dst_hbm, degp_hbm, idx_v, ones_v, zbuf_v, acc_sh, sem):
    core = lax.axis_index("c")
    sid = lax.axis_index("s")
    wid = core * NS + sid

    for k in range(CH // 16):
        ones_v[pl.ds(k * 16, 16)] = jnp.ones((16,), jnp.float32)

    def zb(i, _):
        zbuf_v[pl.ds(i * 16, 16)] = jnp.zeros((16,), jnp.float32)
        return 0

    lax.fori_loop(0, ROWS_PER_TILE // 16, zb, 0)
    pltpu.sync_copy(zbuf_v, acc_sh.at[pl.ds(sid * ROWS_PER_TILE, ROWS_PER_TILE)])
    pltpu.sync_copy(dst_hbm.at[wid], idx_v)
    plsc.subcore_barrier()

    def fire(c, _):
        pltpu.async_copy(ones_v, acc_sh.at[idx_v.at[c]], sem, add=True)
        return 0

    lax.fori_loop(0, NCHUNK, fire, 0)

    def drain(c, _):
        pltpu.make_async_copy(ones_v, acc_sh.at[pl.ds(0, CH)], sem).wait()
        return 0

    lax.fori_loop(0, NCHUNK, drain, 0)
    plsc.subcore_barrier()

    sl = pl.ds(sid * ROWS_PER_TILE, ROWS_PER_TILE)
    pltpu.sync_copy(acc_sh.at[sl], degp_hbm.at[core, sl])


@functools.partial(
    pl.kernel,
    out_type=jax.ShapeDtypeStruct((NC, N_PAD, F), jnp.float32),
    mesh=_mesh,
    scratch_types=[
        pltpu.VMEM((2, SEG, CH), jnp.int32),
        pltpu.VMEM((NCHUNK, CH), jnp.int32),
        pltpu.VMEM((2, CH, F), jnp.float32),
        pltpu.VMEM((ZROWS, F), jnp.float32),
        pltpu.VMEM_SHARED((N_PAD, F), jnp.float32),
        pltpu.SemaphoreType.DMA,
        pltpu.SemaphoreType.DMA,
        pltpu.SemaphoreType.DMA,
        pltpu.SemaphoreType.DMA,
        pltpu.SemaphoreType.DMA,
        pltpu.SemaphoreType.DMA,
        pltpu.SemaphoreType.DMA,
        pltpu.SemaphoreType.DMA,
        pltpu.SemaphoreType.DMA,
    ],
)
def _agg_kernel(hs_hbm, src_hbm, dst_hbm, accp_hbm,
                sbuf_v, idxd_v, rows_v, zbuf_v, acc_sh,
                g00, g01, g10, g11, s0, s1, i0, i1, zsem):
    core = lax.axis_index("c")
    sid = lax.axis_index("s")
    wid = core * NS + sid
    gsem = ((g00, g01), (g10, g11))
    ssem = (s0, s1)
    isem = (i0, i1)

    def issue_i(seg):
        p = seg % 2
        pltpu.async_copy(src_hbm.at[wid, seg], sbuf_v.at[p], isem[p])

    def wait_i(seg):
        p = seg % 2
        pltpu.make_async_copy(src_hbm.at[0, 0], sbuf_v.at[p], isem[p]).wait()

    issue_i(0)
    pltpu.sync_copy(dst_hbm.at[wid], idxd_v)
    _zero_vmem_2d(zbuf_v, ZROWS)

    def zfire(j, _):
        pltpu.async_copy(
            zbuf_v, acc_sh.at[pl.ds(sid * ROWS_PER_TILE + j * ZROWS, ZROWS)], zsem
        )
        return 0

    lax.fori_loop(0, ROWS_PER_TILE // ZROWS, zfire, 0)

    def zdrain(j, _):
        pltpu.make_async_copy(zbuf_v, acc_sh.at[pl.ds(0, ZROWS)], zsem).wait()
        return 0

    lax.fori_loop(0, ROWS_PER_TILE // ZROWS, zdrain, 0)
    plsc.subcore_barrier()

    def issue_g(c, b):
        seg, l = divmod(c, SEG)
        for q in range(2):
            pltpu.async_copy(
                hs_hbm.at[sbuf_v.at[seg % 2, l, pl.ds(q * CH2, CH2)]],
                rows_v.at[b, pl.ds(q * CH2, CH2)],
                gsem[b][q],
            )

    def wait_g(b):
        for q in range(2):
            pltpu.make_async_copy(
                hs_hbm.at[pl.ds(0, CH2)],
                rows_v.at[b, pl.ds(q * CH2, CH2)],
                gsem[b][q],
            ).wait()

    def issue_s(c, b):
        pltpu.async_copy(rows_v.at[b], acc_sh.at[idxd_v.at[c]], ssem[b], add=True)

    def wait_s(b):
        pltpu.make_async_copy(rows_v.at[b], acc_sh.at[pl.ds(0, CH)], ssem[b]).wait()

    wait_i(0)
    issue_g(0, 0)
    for c in range(NCHUNK):
        b = c % 2
        seg, l = divmod(c, SEG)
        if l == 0 and seg + 1 < NSEG:
            issue_i(seg + 1)
        wait_g(b)
        issue_s(c, b)
        if c > 0:
            wait_s(1 - b)
        if c + 1 < NCHUNK:
            if l == SEG - 1:
                wait_i(seg + 1)
            issue_g(c + 1, 1 - b)
    wait_s(1)

    plsc.subcore_barrier()
    sl = pl.ds(sid * ROWS_PER_TILE, ROWS_PER_TILE)
    pltpu.sync_copy(acc_sh.at[sl], accp_hbm.at[core, sl])


RB = 2000
GRID = N // RB


def _dinv(degT_ref):
    d = degT_ref[:, 0:1] + degT_ref[:, 1:2] + 1.0
    return lax.rsqrt(d)


def _tc1_body(x_ref, w_ref, degT_ref, o_ref):
    h = jnp.dot(x_ref[...], w_ref[...], preferred_element_type=jnp.float32)
    o_ref[...] = _dinv(degT_ref) * h


def _tc2_body(acc_ref, hs1_ref, degT_ref, b1_ref, w2_ref, o_ref):
    dinv = _dinv(degT_ref)
    t = dinv * (acc_ref[0] + acc_ref[1] + hs1_ref[...]) + b1_ref[...]
    t = jnp.where(t >= 0, t, 0.01 * t)
    o_ref[...] = dinv * jnp.dot(t, w2_ref[...], preferred_element_type=jnp.float32)


def _tc3_body(acc_ref, hs2_ref, degT_ref, b2_ref, o_ref):
    dinv = _dinv(degT_ref)
    o_ref[...] = dinv * (acc_ref[0] + acc_ref[1] + hs2_ref[...]) + b2_ref[...]


_row_spec = pl.BlockSpec((RB, F), lambda i: (i, 0))
_degT_spec = pl.BlockSpec((RB, 2), lambda i: (i, 0))
_w_spec = pl.BlockSpec((F, F), lambda i: (0, 0))
_b_spec = pl.BlockSpec((1, F), lambda i: (0, 0))
_acc_spec = pl.BlockSpec((NC, RB, F), lambda i: (0, i, 0))

_tc1 = pl.pallas_call(
    _tc1_body,
    grid=(GRID,),
    in_specs=[_row_spec, _w_spec, _degT_spec],
    out_specs=_row_spec,
    out_shape=jax.ShapeDtypeStruct((N, F), jnp.float32),
)

_tc2 = pl.pallas_call(
    _tc2_body,
    grid=(GRID,),
    in_specs=[_acc_spec, _row_spec, _degT_spec, _b_spec, _w_spec],
    out_specs=_row_spec,
    out_shape=jax.ShapeDtypeStruct((N, F), jnp.float32),
)

_tc3 = pl.pallas_call(
    _tc3_body,
    grid=(GRID,),
    in_specs=[_acc_spec, _row_spec, _degT_spec, _b_spec],
    out_specs=_row_spec,
    out_shape=jax.ShapeDtypeStruct((N, F), jnp.float32),
)


def kernel(x, edge_index, W1, b1, W2, b2):
    src = edge_index[0].astype(jnp.int32)
    dst = edge_index[1].astype(jnp.int32)

    npad_e = E_PAD - E
    pad_iota = jnp.arange(npad_e, dtype=jnp.int32)
    src_p = jnp.concatenate([src, pad_iota % N])
    dst_p = jnp.concatenate([dst, N + pad_iota % (N_PAD - N)])
    src_p = src_p.reshape(NW, NSEG, SEG, CH)
    dst_p = dst_p.reshape(NW, NCHUNK, CH)

    degp = _deg_kernel(dst_p)
    degT = degp.T

    b1r = b1.reshape(1, F)
    b2r = b2.reshape(1, F)

    hs1 = _tc1(x, W1, degT)
    acc1 = _agg_kernel(hs1, src_p, dst_p)
    hs2 = _tc2(acc1, hs1, degT, b1r, W2)
    acc2 = _agg_kernel(hs2, src_p, dst_p)
    return _tc3(acc2, hs2, degT, b2r)
